# baseline (speedup 1.0000x reference)
import functools

import jax
import jax.numpy as jnp
from jax import lax
from jax.experimental import pallas as pl
from jax.experimental.pallas import tpu as pltpu
from jax.experimental.pallas import tpu_sc as plsc

_B, _N, _M = 8, 2048, 2048
_ALPHA = 200.0
_L = 16
_TR = 8
_NC, _NS = 2, 16
_NW = _NC * _NS
_TPB = _N // _TR
_NT = _B * _TPB
_TPW = _NT // _NW
_NCH = _M // _L
_BK = 8
_CC = 512


def _allreduce(x, op):
    dnums = lax.GatherDimensionNumbers(
        offset_dims=(), collapsed_slice_dims=(0,), start_index_map=(0,))
    lane = lax.iota(jnp.int32, _L)
    for s in (8, 4, 2, 1):
        perm = jnp.reshape(lane ^ s, (_L, 1))
        y = lax.gather(x, perm, dimension_numbers=dnums, slice_sizes=(1,),
                       mode=lax.GatherScatterMode.PROMISE_IN_BOUNDS)
        x = op(x, y)
    return x


def _voting_body(costs_hbm, ns_hbm, nd_hbm, out_hbm,
                 bufa, bufb, obufa, obufb, zbuf, nsd_v, nsd_s,
                 semia, semib, semoa, semob, semz):
    wid = lax.axis_index("s") * _NC + lax.axis_index("c")
    lane = lax.iota(jnp.int32, _L)
    zero16 = jnp.zeros((_L,), jnp.float32)
    ninf16 = jnp.full((_L,), -jnp.inf, jnp.float32)

    pltpu.sync_copy(ns_hbm, nsd_v.at[pl.ds(0, _B)])
    pltpu.sync_copy(nd_hbm, nsd_v.at[pl.ds(_B, _B)])
    nsd_all = nsd_v[pl.ds(0, 2 * _B)]
    for i in range(2 * _B):
        nsd_s[i] = nsd_all[i]

    @pl.loop(0, _TR * _M // (_BK * _L))
    def _z(i):
        for k in range(_BK):
            j = i * _BK + k
            zbuf[j // _NCH, pl.ds((j % _NCH) * _L, _L)] = zero16

    def _tile_info(t):
        b = jnp.clip(t // _TPB, 0, _B - 1)
        ns = jnp.clip(nsd_s[b], 0, _N)
        nd = jnp.clip(nsd_s[_B + b], 0, _M - 1)
        r0 = (t % _TPB) * _TR
        nact = jnp.where(nd == 0, 0, jnp.clip(ns - r0, 0, _TR))
        return nact, nd

    def _drain_rows(sem, n):
        @pl.loop(0, n)
        def _d(i):
            pltpu.make_async_copy(
                costs_hbm.at[pl.ds(0, 1), :], zbuf.at[pl.ds(0, 1), :],
                sem).wait()

    def _drain_chunks(sem, buf, n):
        @pl.loop(0, n)
        def _d(i):
            pltpu.make_async_copy(
                costs_hbm.at[pl.ds(0, _TR), pl.ds(0, _CC)],
                buf.at[:, pl.ds(0, _CC)], sem).wait()

    def _prefetch(t, ok, buf, sem):
        nact, nd = _tile_info(t)
        go = ok & (nact > 0)
        ncc = (nd + _CC - 1) // _CC
        row0 = t * _TR

        @pl.when(go)
        def _pf():
            @pl.loop(0, ncc)
            def _in(cc):
                pltpu.async_copy(
                    costs_hbm.at[pl.ds(row0, _TR), pl.ds(cc * _CC, _CC)],
                    buf.at[:, pl.ds(cc * _CC, _CC)], sem)

        return jnp.where(go, ncc, 0)

    def _row_softmax(buf, obuf, r, nblk, rtail, pmask, nch_row, hwm,
                     nfull):
        xp = buf[r, pl.ds(nfull * _L, _L)]
        buf[r, pl.ds(nfull * _L, _L)] = jnp.where(pmask, xp, -jnp.inf)

        @pl.loop(0, nblk, init_carry=ninf16)
        def _mx1(ib, mv):
            off = ib * _BK * _L
            for k in range(_BK):
                mv = jnp.maximum(mv, buf[r, pl.ds(off + k * _L, _L)])
            return mv

        @pl.loop(rtail, nch_row, init_carry=_mx1)
        def _mx2(c, mv):
            return jnp.maximum(mv, buf[r, pl.ds(c * _L, _L)])

        mc = _allreduce(_mx2, jnp.maximum) * _ALPHA

        @pl.loop(0, nblk, init_carry=zero16)
        def _p2b(ib, sv):
            off = ib * _BK * _L
            es = []
            for k in range(_BK):
                es.append(
                    jnp.exp(buf[r, pl.ds(off + k * _L, _L)] * _ALPHA - mc))
            for k in range(_BK):
                buf[r, pl.ds(off + k * _L, _L)] = es[k]
                sv = sv + es[k]
            return sv

        @pl.loop(rtail, nch_row, init_carry=_p2b)
        def _p2t(c, sv):
            off = c * _L
            e = jnp.exp(buf[r, pl.ds(off, _L)] * _ALPHA - mc)
            buf[r, pl.ds(off, _L)] = e
            return sv + e

        inv = jnp.full((_L,), 1.0, jnp.float32) / _allreduce(_p2t, jnp.add)

        nblk3 = nch_row // _BK
        rtail3 = nblk3 * _BK

        @pl.loop(0, nblk3)
        def _p3b(ib):
            off = ib * _BK * _L
            for k in range(_BK):
                o = off + k * _L
                obuf[r, pl.ds(o, _L)] = buf[r, pl.ds(o, _L)] * inv

        @pl.loop(rtail3, nch_row)
        def _p3t(c):
            off = c * _L
            obuf[r, pl.ds(off, _L)] = buf[r, pl.ds(off, _L)] * inv

        zhi = jnp.maximum(hwm, nch_row)
        zblk = (zhi - nch_row) // _BK

        @pl.loop(0, zblk)
        def _tzb(ib):
            off = (nch_row + ib * _BK) * _L
            for k in range(_BK):
                obuf[r, pl.ds(off + k * _L, _L)] = zero16

        @pl.loop(nch_row + zblk * _BK, zhi)
        def _tzt(c):
            obuf[r, pl.ds(c * _L, _L)] = zero16

    def _rows8_softmax(buf, obuf, hwm, pmask, nfull, nch_row):
        RF = _TR
        BF = 2
        nblk2 = nch_row // BF
        rtail2 = nblk2 * BF

        for r in range(RF):
            xp = buf[r, pl.ds(nfull * _L, _L)]
            buf[r, pl.ds(nfull * _L, _L)] = jnp.where(pmask, xp, -jnp.inf)

        @pl.loop(0, nblk2, init_carry=(ninf16,) * RF)
        def _mx(ib, mvs):
            off = ib * BF * _L
            out = list(mvs)
            for k in range(BF):
                for r in range(RF):
                    out[r] = jnp.maximum(
                        out[r], buf[r, pl.ds(off + k * _L, _L)])
            return tuple(out)

        @pl.loop(rtail2, nch_row, init_carry=_mx)
        def _mxr(c, mvs):
            off = c * _L
            out = list(mvs)
            for r in range(RF):
                out[r] = jnp.maximum(out[r], buf[r, pl.ds(off, _L)])
            return tuple(out)

        mcs = [_allreduce(mv, jnp.maximum) * _ALPHA for mv in _mxr]

        def _p2_group(r0g, gn):
            @pl.loop(0, nch_row, init_carry=(zero16,) * gn)
            def _p2g(c, svs):
                off = c * _L
                out = list(svs)
                for r4 in range(gn):
                    r = r0g + r4
                    e = jnp.exp(buf[r, pl.ds(off, _L)] * _ALPHA - mcs[r])
                    buf[r, pl.ds(off, _L)] = e
                    out[r4] = out[r4] + e
                return tuple(out)
            return _p2g

        svs_all = _p2_group(0, 3) + _p2_group(3, 3) + _p2_group(6, 2)
        one16 = jnp.full((_L,), 1.0, jnp.float32)
        invs = [one16 / _allreduce(sv, jnp.add) for sv in svs_all]

        @pl.loop(0, nblk2)
        def _p3(ib):
            off = ib * BF * _L
            for k in range(BF):
                for r in range(RF):
                    o = off + k * _L
                    obuf[r, pl.ds(o, _L)] = buf[r, pl.ds(o, _L)] * invs[r]

        @pl.loop(rtail2, nch_row)
        def _p3r(c):
            off = c * _L
            for r in range(RF):
                obuf[r, pl.ds(off, _L)] = buf[r, pl.ds(off, _L)] * invs[r]

        zhi = jnp.maximum(hwm, nch_row)

        @pl.loop(nch_row, zhi)
        def _tz(c):
            off = c * _L
            for r in range(RF):
                obuf[r, pl.ds(off, _L)] = zero16

    def _compute_out(t, buf, obuf, semo, hwm):
        nact, nd = _tile_info(t)
        active = nact > 0
        row0 = t * _TR

        @pl.when(active)
        def _go():
            nfull = nd // _L
            pmask = (nfull * _L + lane) < nd
            nch_row = nfull + jnp.where((nd % _L) > 0, 1, 0)
            nblk = nch_row // _BK
            rtail = nblk * _BK

            @pl.when(nact == _TR)
            def _fast():
                _rows8_softmax(buf, obuf, hwm, pmask, nfull, nch_row)

            @pl.when(nact < _TR)
            def _slow():
                @pl.loop(0, nact)
                def _row(r):
                    _row_softmax(buf, obuf, r, nblk, rtail, pmask, nch_row,
                                 hwm, nfull)

            @pl.when(nact == _TR)
            def _full():
                pltpu.async_copy(obuf, out_hbm.at[pl.ds(row0, _TR), :], semo)

            @pl.when(nact < _TR)
            def _part():
                @pl.loop(0, nact)
                def _rw(r):
                    pltpu.async_copy(
                        obuf.at[pl.ds(r, 1), :],
                        out_hbm.at[pl.ds(row0 + r, 1), :], semo)

        @pl.when(nact == 0)
        def _zt():
            pltpu.async_copy(zbuf, out_hbm.at[pl.ds(row0, _TR), :], semz)

        @pl.when((nact > 0) & (nact < _TR))
        def _zp():
            @pl.loop(nact, _TR)
            def _zw(r):
                pltpu.async_copy(
                    zbuf.at[pl.ds(0, 1), :],
                    out_hbm.at[pl.ds(row0 + r, 1), :], semz)

        nch = (nd + _L - 1) // _L
        hwm = jnp.where(
            active,
            jnp.where(nact == _TR, nch, jnp.maximum(hwm, nch)),
            hwm)
        return jnp.where(active, nact, 0), _TR - nact, hwm

    t0 = wid
    pend_in_a0 = _prefetch(t0, jnp.bool_(True), bufa, semia)
    carry0 = (pend_in_a0, jnp.int32(0), jnp.int32(0), jnp.int32(0),
              jnp.int32(_NCH), jnp.int32(_NCH))

    @pl.loop(0, _TPW // 2, init_carry=carry0)
    def _pair(i, c):
        pend_in_a, pend_out_a, pend_out_b, nzr, hwm_a, hwm_b = c
        ta = wid + (2 * i) * _NW
        tb = ta + _NW
        ta2 = tb + _NW

        _drain_rows(semoa, pend_out_a)
        _drain_chunks(semia, bufa, pend_in_a)
        pend_in_b = _prefetch(tb, jnp.bool_(True), bufb, semib)
        pend_out_a, nz_a, hwm_a = _compute_out(ta, bufa, obufa, semoa, hwm_a)

        _drain_rows(semob, pend_out_b)
        _drain_chunks(semib, bufb, pend_in_b)
        pend_in_a = _prefetch(ta2, i + 1 < _TPW // 2, bufa, semia)
        pend_out_b, nz_b, hwm_b = _compute_out(tb, bufb, obufb, semob, hwm_b)

        return (pend_in_a, pend_out_a, pend_out_b, nzr + nz_a + nz_b,
                hwm_a, hwm_b)

    pend_in_a, pend_out_a, pend_out_b, nzr, _hwa, _hwb = _pair
    _drain_chunks(semia, bufa, pend_in_a)
    _drain_rows(semoa, pend_out_a)
    _drain_rows(semob, pend_out_b)
    _drain_rows(semz, nzr)


_voting_call = functools.partial(
    pl.kernel,
    out_type=jax.ShapeDtypeStruct((_B * _N, _M), jnp.float32),
    mesh=plsc.VectorSubcoreMesh(
        core_axis_name="c", subcore_axis_name="s",
        num_cores=_NC, num_subcores=_NS),
    scratch_types=[
        pltpu.VMEM((_TR, _M), jnp.float32),
        pltpu.VMEM((_TR, _M), jnp.float32),
        pltpu.VMEM((_TR, _M), jnp.float32),
        pltpu.VMEM((_TR, _M), jnp.float32),
        pltpu.VMEM((_TR, _M), jnp.float32),
        pltpu.VMEM((2 * _B,), jnp.int32),
        pltpu.SMEM((2 * _B,), jnp.int32),
        pltpu.SemaphoreType.DMA,
        pltpu.SemaphoreType.DMA,
        pltpu.SemaphoreType.DMA,
        pltpu.SemaphoreType.DMA,
        pltpu.SemaphoreType.DMA,
    ],
)(_voting_body)


@jax.jit
def kernel(costs_batch, ns_src_batch, ns_dst_batch):
    out2 = _voting_call(
        costs_batch.reshape(_B * _N, _M),
        ns_src_batch.astype(jnp.int32),
        ns_dst_batch.astype(jnp.int32),
    )
    return out2.reshape(_B, _N, _M)


# speedup vs baseline: 1.3050x; 1.3050x over previous
import functools

import jax
import jax.numpy as jnp
from jax import lax
from jax.experimental import pallas as pl
from jax.experimental.pallas import tpu as pltpu
from jax.experimental.pallas import tpu_sc as plsc

_B, _N, _M = 8, 2048, 2048
_ALPHA = 200.0
_L = 16
_TR = 8
_NC, _NS = 2, 16
_NW = _NC * _NS
_TPB = _N // _TR
_NT = _B * _TPB
_TPW = _NT // _NW
_NCH = _M // _L
_BK = 8
_CC = 512


def _allreduce(x, op):
    dnums = lax.GatherDimensionNumbers(
        offset_dims=(), collapsed_slice_dims=(0,), start_index_map=(0,))
    lane = lax.iota(jnp.int32, _L)
    for s in (8, 4, 2, 1):
        perm = jnp.reshape(lane ^ s, (_L, 1))
        y = lax.gather(x, perm, dimension_numbers=dnums, slice_sizes=(1,),
                       mode=lax.GatherScatterMode.PROMISE_IN_BOUNDS)
        x = op(x, y)
    return x


def _voting_body(costs_hbm, ns_hbm, nd_hbm, out_hbm,
                 bufa, bufb, obufa, obufb, zbuf, nsd_v, nsd_s,
                 semia, semib, semoa, semob, semz):
    wid = lax.axis_index("s") * _NC + lax.axis_index("c")
    lane = lax.iota(jnp.int32, _L)
    zero16 = jnp.zeros((_L,), jnp.float32)
    ninf16 = jnp.full((_L,), -jnp.inf, jnp.float32)

    pltpu.sync_copy(ns_hbm, nsd_v.at[pl.ds(0, _B)])
    pltpu.sync_copy(nd_hbm, nsd_v.at[pl.ds(_B, _B)])
    nsd_all = nsd_v[pl.ds(0, 2 * _B)]
    for i in range(2 * _B):
        nsd_s[i] = nsd_all[i]

    @pl.loop(0, _TR * _M // (_BK * _L))
    def _z(i):
        for k in range(_BK):
            j = i * _BK + k
            zbuf[j // _NCH, pl.ds((j % _NCH) * _L, _L)] = zero16

    def _tile_info(t):
        b = jnp.clip(t // _TPB, 0, _B - 1)
        ns = jnp.clip(nsd_s[b], 0, _N)
        nd = jnp.clip(nsd_s[_B + b], 0, _M - 1)
        r0 = (t % _TPB) * _TR
        nact = jnp.where(nd == 0, 0, jnp.clip(ns - r0, 0, _TR))
        return nact, nd

    def _drain_rows(sem, n):
        @pl.loop(0, n)
        def _d(i):
            pltpu.make_async_copy(
                costs_hbm.at[pl.ds(0, 1), :], zbuf.at[pl.ds(0, 1), :],
                sem).wait()

    def _drain_chunks(sem, buf, n):
        @pl.loop(0, n)
        def _d(i):
            pltpu.make_async_copy(
                costs_hbm.at[pl.ds(0, _TR), pl.ds(0, _CC)],
                buf.at[:, pl.ds(0, _CC)], sem).wait()

    def _prefetch(t, ok, buf, sem):
        nact, nd = _tile_info(t)
        go = ok & (nact > 0)
        ncc = (nd + _CC - 1) // _CC
        row0 = t * _TR

        @pl.when(go)
        def _pf():
            @pl.loop(0, ncc)
            def _in(cc):
                pltpu.async_copy(
                    costs_hbm.at[pl.ds(row0, _TR), pl.ds(cc * _CC, _CC)],
                    buf.at[:, pl.ds(cc * _CC, _CC)], sem)

        return jnp.where(go, ncc, 0)

    def _row_softmax(buf, obuf, r, nblk, rtail, pmask, nch_row, hwm,
                     nfull):
        xp = buf[r, pl.ds(nfull * _L, _L)]
        buf[r, pl.ds(nfull * _L, _L)] = jnp.where(pmask, xp, -jnp.inf)

        @pl.loop(0, nblk, init_carry=ninf16)
        def _mx1(ib, mv):
            off = ib * _BK * _L
            for k in range(_BK):
                mv = jnp.maximum(mv, buf[r, pl.ds(off + k * _L, _L)])
            return mv

        @pl.loop(rtail, nch_row, init_carry=_mx1)
        def _mx2(c, mv):
            return jnp.maximum(mv, buf[r, pl.ds(c * _L, _L)])

        mc = _allreduce(_mx2, jnp.maximum) * _ALPHA

        @pl.loop(0, nblk, init_carry=zero16)
        def _p2b(ib, sv):
            off = ib * _BK * _L
            es = []
            for k in range(_BK):
                es.append(
                    jnp.exp(buf[r, pl.ds(off + k * _L, _L)] * _ALPHA - mc))
            for k in range(_BK):
                buf[r, pl.ds(off + k * _L, _L)] = es[k]
                sv = sv + es[k]
            return sv

        @pl.loop(rtail, nch_row, init_carry=_p2b)
        def _p2t(c, sv):
            off = c * _L
            e = jnp.exp(buf[r, pl.ds(off, _L)] * _ALPHA - mc)
            buf[r, pl.ds(off, _L)] = e
            return sv + e

        inv = jnp.full((_L,), 1.0, jnp.float32) / _allreduce(_p2t, jnp.add)

        nblk3 = nch_row // _BK
        rtail3 = nblk3 * _BK

        @pl.loop(0, nblk3)
        def _p3b(ib):
            off = ib * _BK * _L
            for k in range(_BK):
                o = off + k * _L
                obuf[r, pl.ds(o, _L)] = buf[r, pl.ds(o, _L)] * inv

        @pl.loop(rtail3, nch_row)
        def _p3t(c):
            off = c * _L
            obuf[r, pl.ds(off, _L)] = buf[r, pl.ds(off, _L)] * inv

        zhi = jnp.maximum(hwm, nch_row)
        zblk = (zhi - nch_row) // _BK

        @pl.loop(0, zblk)
        def _tzb(ib):
            off = (nch_row + ib * _BK) * _L
            for k in range(_BK):
                obuf[r, pl.ds(off + k * _L, _L)] = zero16

        @pl.loop(nch_row + zblk * _BK, zhi)
        def _tzt(c):
            obuf[r, pl.ds(c * _L, _L)] = zero16

    def _rows8_softmax(buf, obuf, hwm, pmask, nfull, nch_row):
        RF = _TR
        BF = 2
        nblk2 = nch_row // BF
        rtail2 = nblk2 * BF

        for r in range(RF):
            xp = buf[r, pl.ds(nfull * _L, _L)]
            buf[r, pl.ds(nfull * _L, _L)] = jnp.where(pmask, xp, -jnp.inf)

        @pl.loop(0, nblk2, init_carry=(ninf16,) * RF)
        def _mx(ib, mvs):
            off = ib * BF * _L
            out = list(mvs)
            for k in range(BF):
                for r in range(RF):
                    out[r] = jnp.maximum(
                        out[r], buf[r, pl.ds(off + k * _L, _L)])
            return tuple(out)

        @pl.loop(rtail2, nch_row, init_carry=_mx)
        def _mxr(c, mvs):
            off = c * _L
            out = list(mvs)
            for r in range(RF):
                out[r] = jnp.maximum(out[r], buf[r, pl.ds(off, _L)])
            return tuple(out)

        mcs = [_allreduce(mv, jnp.maximum) * _ALPHA for mv in _mxr]

        @pl.loop(0, nch_row, init_carry=(zero16,) * RF)
        def _p2(c, svs):
            off = c * _L
            out = list(svs)
            for g in range(2):
                es = []
                for r4 in range(4):
                    r = g * 4 + r4
                    es.append(
                        jnp.exp(buf[r, pl.ds(off, _L)] * _ALPHA - mcs[r]))
                for r4 in range(4):
                    r = g * 4 + r4
                    buf[r, pl.ds(off, _L)] = es[r4]
                    out[r] = out[r] + es[r4]
            return tuple(out)

        svs_all = _p2
        one16 = jnp.full((_L,), 1.0, jnp.float32)
        invs = [one16 / _allreduce(sv, jnp.add) for sv in svs_all]

        @pl.loop(0, nblk2)
        def _p3(ib):
            off = ib * BF * _L
            for k in range(BF):
                for r in range(RF):
                    o = off + k * _L
                    obuf[r, pl.ds(o, _L)] = buf[r, pl.ds(o, _L)] * invs[r]

        @pl.loop(rtail2, nch_row)
        def _p3r(c):
            off = c * _L
            for r in range(RF):
                obuf[r, pl.ds(off, _L)] = buf[r, pl.ds(off, _L)] * invs[r]

        zhi = jnp.maximum(hwm, nch_row)

        @pl.loop(nch_row, zhi)
        def _tz(c):
            off = c * _L
            for r in range(RF):
                obuf[r, pl.ds(off, _L)] = zero16

    def _compute_out(t, buf, obuf, semo, hwm):
        nact, nd = _tile_info(t)
        active = nact > 0
        row0 = t * _TR

        @pl.when(active)
        def _go():
            nfull = nd // _L
            pmask = (nfull * _L + lane) < nd
            nch_row = nfull + jnp.where((nd % _L) > 0, 1, 0)
            nblk = nch_row // _BK
            rtail = nblk * _BK

            @pl.when(nact == _TR)
            def _fast():
                _rows8_softmax(buf, obuf, hwm, pmask, nfull, nch_row)

            @pl.when(nact < _TR)
            def _slow():
                @pl.loop(0, nact)
                def _row(r):
                    _row_softmax(buf, obuf, r, nblk, rtail, pmask, nch_row,
                                 hwm, nfull)

            @pl.when(nact == _TR)
            def _full():
                pltpu.async_copy(obuf, out_hbm.at[pl.ds(row0, _TR), :], semo)

            @pl.when(nact < _TR)
            def _part():
                @pl.loop(0, nact)
                def _rw(r):
                    pltpu.async_copy(
                        obuf.at[pl.ds(r, 1), :],
                        out_hbm.at[pl.ds(row0 + r, 1), :], semo)

        @pl.when(nact == 0)
        def _zt():
            pltpu.async_copy(zbuf, out_hbm.at[pl.ds(row0, _TR), :], semz)

        @pl.when((nact > 0) & (nact < _TR))
        def _zp():
            @pl.loop(nact, _TR)
            def _zw(r):
                pltpu.async_copy(
                    zbuf.at[pl.ds(0, 1), :],
                    out_hbm.at[pl.ds(row0 + r, 1), :], semz)

        nch = (nd + _L - 1) // _L
        hwm = jnp.where(
            active,
            jnp.where(nact == _TR, nch, jnp.maximum(hwm, nch)),
            hwm)
        return jnp.where(active, nact, 0), _TR - nact, hwm

    t0 = wid
    pend_in_a0 = _prefetch(t0, jnp.bool_(True), bufa, semia)
    carry0 = (pend_in_a0, jnp.int32(0), jnp.int32(0), jnp.int32(0),
              jnp.int32(_NCH), jnp.int32(_NCH))

    @pl.loop(0, _TPW // 2, init_carry=carry0)
    def _pair(i, c):
        pend_in_a, pend_out_a, pend_out_b, nzr, hwm_a, hwm_b = c
        ta = wid + (2 * i) * _NW
        tb = ta + _NW
        ta2 = tb + _NW

        _drain_rows(semoa, pend_out_a)
        _drain_chunks(semia, bufa, pend_in_a)
        pend_in_b = _prefetch(tb, jnp.bool_(True), bufb, semib)
        pend_out_a, nz_a, hwm_a = _compute_out(ta, bufa, obufa, semoa, hwm_a)

        _drain_rows(semob, pend_out_b)
        _drain_chunks(semib, bufb, pend_in_b)
        pend_in_a = _prefetch(ta2, i + 1 < _TPW // 2, bufa, semia)
        pend_out_b, nz_b, hwm_b = _compute_out(tb, bufb, obufb, semob, hwm_b)

        return (pend_in_a, pend_out_a, pend_out_b, nzr + nz_a + nz_b,
                hwm_a, hwm_b)

    pend_in_a, pend_out_a, pend_out_b, nzr, _hwa, _hwb = _pair
    _drain_chunks(semia, bufa, pend_in_a)
    _drain_rows(semoa, pend_out_a)
    _drain_rows(semob, pend_out_b)
    _drain_rows(semz, nzr)


_voting_call = functools.partial(
    pl.kernel,
    out_type=jax.ShapeDtypeStruct((_B * _N, _M), jnp.float32),
    mesh=plsc.VectorSubcoreMesh(
        core_axis_name="c", subcore_axis_name="s",
        num_cores=_NC, num_subcores=_NS),
    scratch_types=[
        pltpu.VMEM((_TR, _M), jnp.float32),
        pltpu.VMEM((_TR, _M), jnp.float32),
        pltpu.VMEM((_TR, _M), jnp.float32),
        pltpu.VMEM((_TR, _M), jnp.float32),
        pltpu.VMEM((_TR, _M), jnp.float32),
        pltpu.VMEM((2 * _B,), jnp.int32),
        pltpu.SMEM((2 * _B,), jnp.int32),
        pltpu.SemaphoreType.DMA,
        pltpu.SemaphoreType.DMA,
        pltpu.SemaphoreType.DMA,
        pltpu.SemaphoreType.DMA,
        pltpu.SemaphoreType.DMA,
    ],
)(_voting_body)


@jax.jit
def kernel(costs_batch, ns_src_batch, ns_dst_batch):
    out2 = _voting_call(
        costs_batch.reshape(_B * _N, _M),
        ns_src_batch.astype(jnp.int32),
        ns_dst_batch.astype(jnp.int32),
    )
    return out2.reshape(_B, _N, _M)
